# baseline (speedup 1.0000x reference)
import functools

import jax
import jax.numpy as jnp
from jax.experimental import pallas as pl
from jax.experimental.pallas import tpu as pltpu

_NSLOT = 6
_PREF = 3
_LCHUNK = 1024


def _lane_chunks(hw):
    chunks = []
    off = 0
    while off < hw:
        w = min(_LCHUNK, hw - off)
        chunks.append((off, w))
        off += w
    return chunks


def _se_kernel(x_hbm, w1_ref, b1_ref, w2_ref, b2_ref, y_hbm,
               xbuf, in_sem, out_sem, *, inv_hw):
    n_b, c, hw = x_hbm.shape
    chunks = _lane_chunks(hw)

    def start_in(n, slot):
        for q, (off, w) in enumerate(chunks):
            pltpu.make_async_copy(
                x_hbm.at[n, :, pl.ds(off, w)],
                xbuf.at[slot, :, pl.ds(off, w)],
                in_sem.at[slot, q]).start()

    def wait_in(slot):
        for q, (off, w) in enumerate(chunks):
            pltpu.make_async_copy(
                x_hbm.at[0, :, pl.ds(off, w)],
                xbuf.at[slot, :, pl.ds(off, w)],
                in_sem.at[slot, q]).wait()

    def start_out(n, slot):
        for q, (off, w) in enumerate(chunks):
            pltpu.make_async_copy(
                xbuf.at[slot, :, pl.ds(off, w)],
                y_hbm.at[n, :, pl.ds(off, w)],
                out_sem.at[slot, q]).start(priority=1)

    def wait_out(slot):
        for q, (off, w) in enumerate(chunks):
            pltpu.make_async_copy(
                xbuf.at[slot, :, pl.ds(off, w)],
                y_hbm.at[0, :, pl.ds(off, w)],
                out_sem.at[slot, q]).wait()

    for n in range(_PREF):
        start_in(n, n % _NSLOT)

    def body(n, _):
        slot = jax.lax.rem(n, _NSLOT)

        @pl.when(n + _PREF < n_b)
        def _():
            tgt = jax.lax.rem(n + _PREF, _NSLOT)

            @pl.when(n + _PREF >= _NSLOT)
            def _():
                wait_out(tgt)
            start_in(n + _PREF, tgt)

        wait_in(slot)
        x = xbuf[slot]
        pooled = jnp.sum(x, axis=-1, keepdims=True) * inv_hw
        h = jnp.dot(w1_ref[...], pooled,
                    preferred_element_type=jnp.float32)
        h = jnp.maximum(h + b1_ref[...], 0.0)
        z = jnp.dot(w2_ref[...], h,
                    preferred_element_type=jnp.float32)
        g = jax.nn.sigmoid(z + b2_ref[...])
        xbuf[slot] = x * g
        start_out(n, slot)
        return ()

    jax.lax.fori_loop(0, n_b, body, (), unroll=False)

    for k in range(min(_NSLOT, n_b)):
        wait_out((n_b - 1 - k) % _NSLOT)


def kernel(x, w_reduce, b_reduce, w_expand, b_expand):
    N, C, H, W = x.shape
    hw = H * W
    cr = w_reduce.shape[0]

    xf = x.reshape(N, C, hw)
    w1 = w_reduce.astype(jnp.float32)
    b1 = b_reduce.astype(jnp.float32)
    w2 = w_expand.astype(jnp.float32)
    b2 = b_expand.astype(jnp.float32)

    y = pl.pallas_call(
        functools.partial(_se_kernel, inv_hw=1.0 / float(hw)),
        out_shape=jax.ShapeDtypeStruct((N, C, hw), x.dtype),
        in_specs=[
            pl.BlockSpec(memory_space=pltpu.MemorySpace.HBM),
            pl.BlockSpec((cr, C), lambda: (0, 0)),
            pl.BlockSpec((cr, 1), lambda: (0, 0)),
            pl.BlockSpec((C, cr), lambda: (0, 0)),
            pl.BlockSpec((C, 1), lambda: (0, 0)),
        ],
        out_specs=pl.BlockSpec(memory_space=pltpu.MemorySpace.HBM),
        scratch_shapes=[
            pltpu.VMEM((_NSLOT, C, hw), jnp.float32),
            pltpu.SemaphoreType.DMA((_NSLOT, len(_lane_chunks(hw)))),
            pltpu.SemaphoreType.DMA((_NSLOT, len(_lane_chunks(hw)))),
        ],
        cost_estimate=pl.CostEstimate(
            flops=int(2 * N * C * hw + 4 * N * C * cr),
            transcendentals=int(N * C),
            bytes_accessed=int(2 * xf.size * x.dtype.itemsize
                               + (w1.size + b1.size + w2.size + b2.size) * 4),
        ),
    )(xf, w1, b1, w2, b2)

    return y.reshape(N, C, H, W)
